# Initial kernel scaffold; baseline (speedup 1.0000x reference)
import jax
import jax.numpy as jnp
from jax.experimental import pallas as pl


def kernel(input_batch, table):
    raise NotImplementedError("write your pallas kernel here")


# speedup vs baseline: 4.0672x; 4.0672x over previous
import jax
import jax.numpy as jnp
from jax import lax
from jax.experimental import pallas as pl
from jax.experimental.pallas import tpu as pltpu
from jax.experimental.pallas import tpu_sc as plsc

BATCH = 4096
SEQ = 50
DIM = 64
PADDING_IDX = 0

NUM_CORES = 2
NUM_SUBCORES = 16
NUM_WORKERS = NUM_CORES * NUM_SUBCORES

TOTAL = BATCH * SEQ
CHUNK = 128
ROWS_PER_WORKER = TOTAL // NUM_WORKERS
CHUNKS_PER_WORKER = ROWS_PER_WORKER // CHUNK
LANES = 16
GROUPS = CHUNK // LANES
COLV = DIM // LANES


def _lookup_body(table_hbm, idx_hbm, out_hbm, idx_v, rows_v, sem):
    wid = lax.axis_index("s") * NUM_CORES + lax.axis_index("c")
    pltpu.sync_copy(idx_hbm.at[pl.ds(wid * ROWS_PER_WORKER, ROWS_PER_WORKER)],
                    idx_v)

    def chunk_body(j, carry):
        pltpu.async_copy(
            table_hbm.at[idx_v.at[pl.ds(j * CHUNK, CHUNK)]], rows_v, sem
        ).wait()

        vmin = idx_v[pl.ds(j * CHUNK, LANES)]
        for g in range(1, GROUPS):
            vmin = jnp.minimum(vmin,
                               idx_v[pl.ds(j * CHUNK + g * LANES, LANES)])
        smin = vmin[0]
        for i in range(1, LANES):
            smin = jnp.minimum(smin, vmin[i])

        @pl.when(smin == PADDING_IDX)
        def _fix():
            def grp_body(g, c2):
                idx16 = idx_v[pl.ds(j * CHUNK + g * LANES, LANES)]
                for i in range(LANES):

                    @pl.when(idx16[i] == PADDING_IDX)
                    def _zero(i=i):
                        r = g * LANES + i
                        for c in range(COLV):
                            rows_v[r, pl.ds(c * LANES, LANES)] = jnp.zeros(
                                (LANES,), jnp.float32)

                return c2

            lax.fori_loop(0, GROUPS, grp_body, 0)

        pltpu.sync_copy(
            rows_v, out_hbm.at[pl.ds(wid * ROWS_PER_WORKER + j * CHUNK, CHUNK)])
        return carry

    lax.fori_loop(0, CHUNKS_PER_WORKER, chunk_body, 0)


_lookup = pl.kernel(
    _lookup_body,
    out_type=jax.ShapeDtypeStruct((TOTAL, DIM), jnp.float32),
    mesh=plsc.VectorSubcoreMesh(core_axis_name="c", subcore_axis_name="s"),
    compiler_params=pltpu.CompilerParams(use_tc_tiling_on_sc=False),
    scratch_types=[
        pltpu.VMEM((ROWS_PER_WORKER,), jnp.int32),
        pltpu.VMEM((CHUNK, DIM), jnp.float32),
        pltpu.SemaphoreType.DMA,
    ],
)


def kernel(input_batch, table):
    idx = input_batch.reshape(TOTAL).astype(jnp.int32)
    out = _lookup(table, idx)
    return out.reshape(BATCH, SEQ, DIM)


# speedup vs baseline: 4.6219x; 1.1364x over previous
import jax
import jax.numpy as jnp
from jax import lax
from jax.experimental import pallas as pl
from jax.experimental.pallas import tpu as pltpu
from jax.experimental.pallas import tpu_sc as plsc

BATCH = 4096
SEQ = 50
DIM = 64
PADDING_IDX = 0

NUM_CORES = 2
NUM_SUBCORES = 16
NUM_WORKERS = NUM_CORES * NUM_SUBCORES

TOTAL = BATCH * SEQ
CHUNK = 128
ROWS_PER_WORKER = TOTAL // NUM_WORKERS
CHUNKS_PER_WORKER = ROWS_PER_WORKER // CHUNK
LANES = 16
GROUPS = CHUNK // LANES
COLV = DIM // LANES


NBUF = 10
ROUNDS = CHUNKS_PER_WORKER // NBUF


def _lookup_body(table_hbm, idx_hbm, out_hbm, idx_v, rows_v, *sems):
    gsems, ssems = sems[:NBUF], sems[NBUF:]
    wid = lax.axis_index("s") * NUM_CORES + lax.axis_index("c")
    base = wid * ROWS_PER_WORKER
    pltpu.sync_copy(idx_hbm.at[pl.ds(base, ROWS_PER_WORKER)], idx_v)

    def gather_desc(j, b):
        return pltpu.make_async_copy(
            table_hbm.at[idx_v.at[pl.ds(j * CHUNK, CHUNK)]],
            rows_v.at[b], gsems[b])

    def store_desc(j, b):
        return pltpu.make_async_copy(
            rows_v.at[b], out_hbm.at[pl.ds(base + j * CHUNK, CHUNK)], ssems[b])

    def fixup(j, b):
        vmin = idx_v[pl.ds(j * CHUNK, LANES)]
        for g in range(1, GROUPS):
            vmin = jnp.minimum(vmin,
                               idx_v[pl.ds(j * CHUNK + g * LANES, LANES)])
        smin = vmin[0]
        for i in range(1, LANES):
            smin = jnp.minimum(smin, vmin[i])

        @pl.when(smin == PADDING_IDX)
        def _fix():
            def grp_body(g, c2):
                idx16 = idx_v[pl.ds(j * CHUNK + g * LANES, LANES)]
                for i in range(LANES):

                    @pl.when(idx16[i] == PADDING_IDX)
                    def _zero(i=i):
                        r = g * LANES + i
                        for c in range(COLV):
                            rows_v[b, r, pl.ds(c * LANES, LANES)] = jnp.zeros(
                                (LANES,), jnp.float32)

                return c2

            lax.fori_loop(0, GROUPS, grp_body, 0)

    for b in range(NBUF):
        gather_desc(b, b).start()

    def round_body(t, carry):
        for b in range(NBUF):
            j = t * NBUF + b
            gather_desc(j, b).wait()
            fixup(j, b)
            store_desc(j, b).start()

        @pl.when(t < ROUNDS - 1)
        def _issue_next():
            for b in range(NBUF):
                j = t * NBUF + b
                store_desc(j, b).wait()
                gather_desc(j + NBUF, b).start()

        return carry

    lax.fori_loop(0, ROUNDS, round_body, 0)

    for b in range(NBUF):
        store_desc((ROUNDS - 1) * NBUF + b, b).wait()


_lookup = pl.kernel(
    _lookup_body,
    out_type=jax.ShapeDtypeStruct((TOTAL, DIM), jnp.float32),
    mesh=plsc.VectorSubcoreMesh(core_axis_name="c", subcore_axis_name="s"),
    compiler_params=pltpu.CompilerParams(use_tc_tiling_on_sc=False),
    scratch_types=[
        pltpu.VMEM((ROWS_PER_WORKER,), jnp.int32),
        pltpu.VMEM((NBUF, CHUNK, DIM), jnp.float32),
    ] + [pltpu.SemaphoreType.DMA] * (2 * NBUF),
)


def kernel(input_batch, table):
    idx = input_batch.reshape(TOTAL).astype(jnp.int32)
    out = _lookup(table, idx)
    return out.reshape(BATCH, SEQ, DIM)
